# baseline (speedup 1.0000x reference)
import functools

import jax
import jax.numpy as jnp
from jax import lax
from jax.experimental import pallas as pl
from jax.experimental.pallas import tpu as pltpu
from jax.experimental.pallas import tpu_sc as plsc

VOCAB = 1000
NTOK = 20480
NC, NS, L = 2, 16, 16
NW = NC * NS
TPW = NTOK // NW
CHUNK = 16
NCHUNK = TPW // CHUNK


def _sc_mesh():
    return plsc.VectorSubcoreMesh(
        core_axis_name="c", subcore_axis_name="s", num_cores=NC, num_subcores=NS
    )


def _sc_gather(idx2, tgt2, emb, emb_flat):
    GCH = 128

    @functools.partial(
        pl.kernel,
        mesh=_sc_mesh(),
        compiler_params=pltpu.CompilerParams(use_tc_tiling_on_sc=False),
        out_type=(
            jax.ShapeDtypeStruct((NTOK, VOCAB), jnp.float32),
            jax.ShapeDtypeStruct((NW * L,), jnp.float32),
        ),
        scratch_types=[
            pltpu.VMEM((NCHUNK, CHUNK), jnp.int32),
            pltpu.VMEM((NCHUNK, CHUNK), jnp.int32),
            pltpu.VMEM((TPW,), jnp.int32),
            pltpu.VMEM((TPW,), jnp.float32),
            pltpu.VMEM((L,), jnp.float32),
            pltpu.VMEM((CHUNK, VOCAB), jnp.float32),
            pltpu.VMEM((CHUNK, VOCAB), jnp.float32),
            pltpu.SemaphoreType.DMA,
            pltpu.SemaphoreType.DMA,
            pltpu.SemaphoreType.DMA,
            pltpu.SemaphoreType.DMA,
            pltpu.SemaphoreType.DMA,
        ],
    )
    def k(idx_hbm, tgt_hbm, table_hbm, flat_hbm, out_hbm, part_hbm,
          idx_v, tgt_v, fidx_v, vals_v, pv, buf0, buf1, gs0, gs1, os0, os1, fsem):
        wid = lax.axis_index("s") * NC + lax.axis_index("c")
        pltpu.sync_copy(idx_hbm.at[pl.ds(wid * NCHUNK, NCHUNK)], idx_v)
        pltpu.sync_copy(tgt_hbm.at[pl.ds(wid * NCHUNK, NCHUNK)], tgt_v)
        bufs, gsems, osems = (buf0, buf1), (gs0, gs1), (os0, os1)

        def fbody(c, _):
            fidx_v[pl.ds(c * CHUNK, CHUNK)] = idx_v[c, :] * VOCAB + tgt_v[c, :]
            return 0

        lax.fori_loop(0, NCHUNK, fbody, 0)
        for c in range(TPW // GCH):
            pltpu.async_copy(
                flat_hbm.at[fidx_v.at[pl.ds(c * GCH, GCH)]],
                vals_v.at[pl.ds(c * GCH, GCH)],
                fsem,
            )

        def gather(c):
            b = c & 1
            return pltpu.make_async_copy(table_hbm.at[idx_v.at[c]], bufs[b], gsems[b])

        def put(c):
            b = c & 1
            return pltpu.make_async_copy(
                bufs[b], out_hbm.at[pl.ds(wid * TPW + c * CHUNK, CHUNK)], osems[b]
            )

        gather(0).start()
        for c in range(NCHUNK):
            if c + 1 < NCHUNK:
                if c >= 1:
                    put(c - 1).wait()
                gather(c + 1).start()
            gather(c).wait()
            put(c).start()
        put(NCHUNK - 2).wait()
        put(NCHUNK - 1).wait()

        for c in range(TPW // GCH):
            pltpu.make_async_copy(
                flat_hbm.at[fidx_v.at[pl.ds(c * GCH, GCH)]],
                vals_v.at[pl.ds(c * GCH, GCH)],
                fsem,
            ).wait()

        def sbody(c, acc):
            return acc + vals_v[pl.ds(c * L, L)]

        acc = lax.fori_loop(0, TPW // L, sbody, jnp.zeros((L,), jnp.float32))
        pv[...] = acc
        pltpu.sync_copy(pv, part_hbm.at[pl.ds(wid * L, L)])

    return k(idx2, tgt2, emb, emb_flat)


def _tc_lse_dot(emb, idx_row):
    CH = 2048

    def body(emb_ref, idx_ref, out_ref):
        x = emb_ref[...]
        m = jnp.max(x, axis=1, keepdims=True)
        s = jnp.sum(jnp.exp(x - m), axis=1, keepdims=True)
        lse = jnp.log(s) + m
        riota = lax.broadcasted_iota(jnp.int32, (VOCAB, CH), 0)

        def cbody(c, acc):
            ids = idx_ref[:, pl.ds(c * CH, CH)]
            cnt = jnp.sum(jnp.where(riota == ids, 1.0, 0.0), axis=1, keepdims=True)
            return acc + cnt

        counts = lax.fori_loop(0, NTOK // CH, cbody, jnp.zeros((VOCAB, 1), jnp.float32))
        out_ref[...] = jnp.sum(counts * lse).reshape(1, 1)

    return pl.pallas_call(
        body,
        out_shape=jax.ShapeDtypeStruct((1, 1), jnp.float32),
    )(emb, idx_row)


def kernel(idx, targets, emb):
    idx2 = idx.reshape(NW * NCHUNK, CHUNK)
    tgt2 = targets.reshape(NW * NCHUNK, CHUNK)
    emb_flat = jnp.pad(emb.reshape(-1), (0, 8))
    logits2, partials = _sc_gather(idx2, tgt2, emb, emb_flat)
    lse_sum = _tc_lse_dot(emb, idx.reshape(1, NTOK))
    loss = (lse_sum[0, 0] - jnp.sum(partials)) / NTOK
    return (logits2, loss)


# speedup vs baseline: 1.0149x; 1.0149x over previous
import functools

import jax
import jax.numpy as jnp
from jax import lax
from jax.experimental import pallas as pl
from jax.experimental.pallas import tpu as pltpu
from jax.experimental.pallas import tpu_sc as plsc

VOCAB = 1000
NTOK = 20480
NC, NS, L = 2, 16, 16
NW = NC * NS
TPW = NTOK // NW
CHUNK = 16
NCHUNK = TPW // CHUNK


def _sc_mesh():
    return plsc.VectorSubcoreMesh(
        core_axis_name="c", subcore_axis_name="s", num_cores=NC, num_subcores=NS
    )


def _sc_gather(idx2, emb):
    @functools.partial(
        pl.kernel,
        mesh=_sc_mesh(),
        compiler_params=pltpu.CompilerParams(use_tc_tiling_on_sc=False),
        out_type=jax.ShapeDtypeStruct((NTOK, VOCAB), jnp.float32),
        scratch_types=[
            pltpu.VMEM((NCHUNK, CHUNK), jnp.int32),
            pltpu.VMEM((CHUNK, VOCAB), jnp.float32),
            pltpu.VMEM((CHUNK, VOCAB), jnp.float32),
            pltpu.SemaphoreType.DMA,
            pltpu.SemaphoreType.DMA,
            pltpu.SemaphoreType.DMA,
            pltpu.SemaphoreType.DMA,
        ],
    )
    def k(idx_hbm, table_hbm, out_hbm, idx_v, buf0, buf1, gs0, gs1, os0, os1):
        wid = lax.axis_index("s") * NC + lax.axis_index("c")
        pltpu.sync_copy(idx_hbm.at[pl.ds(wid * NCHUNK, NCHUNK)], idx_v)
        bufs, gsems, osems = (buf0, buf1), (gs0, gs1), (os0, os1)

        def gather(c):
            b = c & 1
            return pltpu.make_async_copy(table_hbm.at[idx_v.at[c]], bufs[b], gsems[b])

        def put(c):
            b = c & 1
            return pltpu.make_async_copy(
                bufs[b], out_hbm.at[pl.ds(wid * TPW + c * CHUNK, CHUNK)], osems[b]
            )

        gather(0).start()
        for c in range(NCHUNK):
            if c + 1 < NCHUNK:
                if c >= 1:
                    put(c - 1).wait()
                gather(c + 1).start()
            gather(c).wait()
            put(c).start()
        put(NCHUNK - 2).wait()
        put(NCHUNK - 1).wait()

    return k(idx2, emb)


def _sc_tgt_sum(tgt2, logits_flat):
    GCH = 128

    @functools.partial(
        pl.kernel,
        mesh=_sc_mesh(),
        compiler_params=pltpu.CompilerParams(use_tc_tiling_on_sc=False),
        out_type=jax.ShapeDtypeStruct((NW * L,), jnp.float32),
        scratch_types=[
            pltpu.VMEM((NCHUNK, CHUNK), jnp.int32),
            pltpu.VMEM((TPW,), jnp.int32),
            pltpu.VMEM((TPW,), jnp.float32),
            pltpu.VMEM((L,), jnp.float32),
            pltpu.SemaphoreType.DMA,
        ],
    )
    def k(tgt_hbm, flat_hbm, part_hbm, tgt_v, fidx_v, vals_v, pv, gsem):
        wid = lax.axis_index("s") * NC + lax.axis_index("c")
        rbase = wid * NCHUNK
        pltpu.sync_copy(tgt_hbm.at[pl.ds(rbase, NCHUNK)], tgt_v)

        def fbody(c, _):
            tbase = (wid * TPW + c * CHUNK) * VOCAB
            toff = lax.iota(jnp.int32, L) * VOCAB + tbase
            fidx_v[pl.ds(c * CHUNK, CHUNK)] = toff + tgt_v[c, :]
            return 0

        lax.fori_loop(0, NCHUNK, fbody, 0)

        for c in range(TPW // GCH):
            pltpu.async_copy(
                flat_hbm.at[fidx_v.at[pl.ds(c * GCH, GCH)]],
                vals_v.at[pl.ds(c * GCH, GCH)],
                gsem,
            )
        for c in range(TPW // GCH):
            pltpu.make_async_copy(
                flat_hbm.at[fidx_v.at[pl.ds(c * GCH, GCH)]],
                vals_v.at[pl.ds(c * GCH, GCH)],
                gsem,
            ).wait()

        def sbody(c, acc):
            return acc + vals_v[pl.ds(c * L, L)]

        acc = lax.fori_loop(0, TPW // L, sbody, jnp.zeros((L,), jnp.float32))
        pv[...] = acc
        pltpu.sync_copy(pv, part_hbm.at[pl.ds(wid * L, L)])

    return k(tgt2, logits_flat)


def _tc_lse_dot(emb, idx_row):
    CH = 2048

    def body(emb_ref, idx_ref, out_ref):
        x = emb_ref[...]
        m = jnp.max(x, axis=1, keepdims=True)
        s = jnp.sum(jnp.exp(x - m), axis=1, keepdims=True)
        lse = jnp.log(s) + m
        riota = lax.broadcasted_iota(jnp.int32, (VOCAB, CH), 0)

        def cbody(c, acc):
            ids = idx_ref[:, pl.ds(c * CH, CH)]
            cnt = jnp.sum(jnp.where(riota == ids, 1.0, 0.0), axis=1, keepdims=True)
            return acc + cnt

        counts = lax.fori_loop(0, NTOK // CH, cbody, jnp.zeros((VOCAB, 1), jnp.float32))
        out_ref[...] = jnp.sum(counts * lse).reshape(1, 1)

    return pl.pallas_call(
        body,
        out_shape=jax.ShapeDtypeStruct((1, 1), jnp.float32),
    )(emb, idx_row)


def kernel(idx, targets, emb):
    idx2 = idx.reshape(NW * NCHUNK, CHUNK)
    tgt2 = targets.reshape(NW * NCHUNK, CHUNK)
    logits2 = _sc_gather(idx2, emb)
    partials = _sc_tgt_sum(tgt2, logits2.reshape(-1))
    lse_sum = _tc_lse_dot(emb, idx.reshape(1, NTOK))
    loss = (lse_sum[0, 0] - jnp.sum(partials)) / NTOK
    return (logits2, loss)


# speedup vs baseline: 1.0172x; 1.0023x over previous
import functools

import jax
import jax.numpy as jnp
from jax import lax
from jax.experimental import pallas as pl
from jax.experimental.pallas import tpu as pltpu
from jax.experimental.pallas import tpu_sc as plsc

VOCAB = 1000
NTOK = 20480
NC, NS, L = 2, 16, 16
NW = NC * NS
TPW = NTOK // NW
CHUNK = 16
NCHUNK = TPW // CHUNK


def _sc_mesh():
    return plsc.VectorSubcoreMesh(
        core_axis_name="c", subcore_axis_name="s", num_cores=NC, num_subcores=NS
    )


def _sc_gather(idx2, emb):
    @functools.partial(
        pl.kernel,
        mesh=_sc_mesh(),
        compiler_params=pltpu.CompilerParams(use_tc_tiling_on_sc=False),
        out_type=jax.ShapeDtypeStruct((NTOK, VOCAB), jnp.float32),
        scratch_types=[
            pltpu.VMEM((NCHUNK, CHUNK), jnp.int32),
            pltpu.VMEM((CHUNK, VOCAB), jnp.float32),
            pltpu.VMEM((CHUNK, VOCAB), jnp.float32),
            pltpu.SemaphoreType.DMA,
            pltpu.SemaphoreType.DMA,
            pltpu.SemaphoreType.DMA,
            pltpu.SemaphoreType.DMA,
        ],
    )
    def k(idx_hbm, table_hbm, out_hbm, idx_v, buf0, buf1, gs0, gs1, os0, os1):
        wid = lax.axis_index("s") * NC + lax.axis_index("c")
        pltpu.sync_copy(idx_hbm.at[pl.ds(wid * NCHUNK, NCHUNK)], idx_v)
        bufs, gsems, osems = (buf0, buf1), (gs0, gs1), (os0, os1)

        def gather(c):
            b = c & 1
            return pltpu.make_async_copy(table_hbm.at[idx_v.at[c]], bufs[b], gsems[b])

        def put(c):
            b = c & 1
            return pltpu.make_async_copy(
                bufs[b], out_hbm.at[pl.ds(wid * TPW + c * CHUNK, CHUNK)], osems[b]
            )

        gather(0).start()
        for c in range(NCHUNK):
            if c + 1 < NCHUNK:
                if c >= 1:
                    put(c - 1).wait()
                gather(c + 1).start()
            gather(c).wait()
            put(c).start()
        put(NCHUNK - 2).wait()
        put(NCHUNK - 1).wait()

    return k(idx2, emb)


def _sc_tgt_sum(idx2, tgt2, emb_flat):
    GCH = 128

    @functools.partial(
        pl.kernel,
        mesh=_sc_mesh(),
        compiler_params=pltpu.CompilerParams(use_tc_tiling_on_sc=False),
        out_type=jax.ShapeDtypeStruct((NW * L,), jnp.float32),
        scratch_types=[
            pltpu.VMEM((NCHUNK, CHUNK), jnp.int32),
            pltpu.VMEM((NCHUNK, CHUNK), jnp.int32),
            pltpu.VMEM((TPW,), jnp.int32),
            pltpu.VMEM((TPW,), jnp.float32),
            pltpu.VMEM((L,), jnp.float32),
            pltpu.SemaphoreType.DMA,
        ],
    )
    def k(idx_hbm, tgt_hbm, flat_hbm, part_hbm, idx_v, tgt_v, fidx_v, vals_v, pv, gsem):
        wid = lax.axis_index("s") * NC + lax.axis_index("c")
        rbase = wid * NCHUNK
        pltpu.sync_copy(idx_hbm.at[pl.ds(rbase, NCHUNK)], idx_v)
        pltpu.sync_copy(tgt_hbm.at[pl.ds(rbase, NCHUNK)], tgt_v)

        def fbody(c, _):
            fidx_v[pl.ds(c * CHUNK, CHUNK)] = idx_v[c, :] * VOCAB + tgt_v[c, :]
            return 0

        lax.fori_loop(0, NCHUNK, fbody, 0)

        for c in range(TPW // GCH):
            pltpu.async_copy(
                flat_hbm.at[fidx_v.at[pl.ds(c * GCH, GCH)]],
                vals_v.at[pl.ds(c * GCH, GCH)],
                gsem,
            )
        for c in range(TPW // GCH):
            pltpu.make_async_copy(
                flat_hbm.at[fidx_v.at[pl.ds(c * GCH, GCH)]],
                vals_v.at[pl.ds(c * GCH, GCH)],
                gsem,
            ).wait()

        def sbody(c, acc):
            return acc + vals_v[pl.ds(c * L, L)]

        acc = lax.fori_loop(0, TPW // L, sbody, jnp.zeros((L,), jnp.float32))
        pv[...] = acc
        pltpu.sync_copy(pv, part_hbm.at[pl.ds(wid * L, L)])

    return k(idx2, tgt2, emb_flat)


def _tc_lse_dot(emb, idx_row):
    CH = 2048

    def body(emb_ref, idx_ref, out_ref):
        x = emb_ref[...]
        m = jnp.max(x, axis=1, keepdims=True)
        s = jnp.sum(jnp.exp(x - m), axis=1, keepdims=True)
        lse = jnp.log(s) + m
        riota = lax.broadcasted_iota(jnp.int32, (VOCAB, CH), 0)

        def cbody(c, acc):
            ids = idx_ref[:, pl.ds(c * CH, CH)]
            cnt = jnp.sum(jnp.where(riota == ids, 1.0, 0.0), axis=1, keepdims=True)
            return acc + cnt

        counts = lax.fori_loop(0, NTOK // CH, cbody, jnp.zeros((VOCAB, 1), jnp.float32))
        out_ref[...] = jnp.sum(counts * lse).reshape(1, 1)

    return pl.pallas_call(
        body,
        out_shape=jax.ShapeDtypeStruct((1, 1), jnp.float32),
    )(emb, idx_row)


def kernel(idx, targets, emb):
    idx2 = idx.reshape(NW * NCHUNK, CHUNK)
    tgt2 = targets.reshape(NW * NCHUNK, CHUNK)
    logits2 = _sc_gather(idx2, emb)
    partials = _sc_tgt_sum(idx2, tgt2, emb.reshape(-1))
    lse_sum = _tc_lse_dot(emb, idx.reshape(1, NTOK))
    loss = (lse_sum[0, 0] - jnp.sum(partials)) / NTOK
    return (logits2, loss)
